# Initial kernel scaffold; baseline (speedup 1.0000x reference)
import jax
import jax.numpy as jnp
from jax.experimental import pallas as pl


def kernel(xyz, W1, W2, W3):
    raise NotImplementedError("write your pallas kernel here")


# speedup vs baseline: 17.1590x; 17.1590x over previous
import functools

import jax
import jax.numpy as jnp
from jax import lax
from jax.experimental import pallas as pl
from jax.experimental.pallas import tpu as pltpu
from jax.experimental.pallas import tpu_sc as plsc

B = 4
N = 4096
COUT = 128
KSEL = 17
KNN = 16
CH = 64
NCH = N // CH
QT = 256
NB = 2048
NTEC = 32
RPT = (B * N) // NTEC
TPB = N // RPT
SVMAX = N + 64


def _dist_body(q_ref, c_ref, dist_ref, cm_ref, thr_ref):
    q = q_ref[0]
    c = c_ref[0]
    d0 = c[0:1, :] - q[:, 0:1]
    d1 = c[1:2, :] - q[:, 1:2]
    d2 = c[2:3, :] - q[:, 2:3]
    dist = (d0 * d0 + d1 * d1) + d2 * d2
    dist_ref[0] = dist
    cms = [jnp.min(dist[:, i * CH:(i + 1) * CH], axis=1, keepdims=True)
           for i in range(NCH)]
    cm = jnp.concatenate(cms, axis=1)
    cm_ref[0] = cm
    iota = lax.broadcasted_iota(jnp.int32, cm.shape, 1)
    work = cm
    m = None
    for _ in range(KSEL):
        m = jnp.min(work, axis=1, keepdims=True)
        sel = jnp.min(jnp.where(work == m, iota, NCH), axis=1, keepdims=True)
        work = jnp.where(iota == sel, jnp.float32(jnp.inf), work)
    thr_ref[0] = m


_dist_call = pl.pallas_call(
    _dist_body,
    grid=(B, N // QT),
    in_specs=[
        pl.BlockSpec((1, QT, 3), lambda b, t: (b, t, 0)),
        pl.BlockSpec((1, 3, N), lambda b, t: (b, 0, 0)),
    ],
    out_specs=[
        pl.BlockSpec((1, QT, N), lambda b, t: (b, t, 0)),
        pl.BlockSpec((1, QT, NCH), lambda b, t: (b, t, 0)),
        pl.BlockSpec((1, QT, 1), lambda b, t: (b, t, 0)),
    ],
    out_shape=[
        jax.ShapeDtypeStruct((B, N, N), jnp.float32),
        jax.ShapeDtypeStruct((B, N, NCH), jnp.float32),
        jax.ShapeDtypeStruct((B, N, 1), jnp.float32),
    ],
)


def _sc_body(dist_hbm, thr_hbm, cm_hbm, xc_hbm, yc_hbm, zc_hbm,
             ox_hbm, oy_hbm, oz_hbm,
             xl, yl, zl, thrl, cml, db0, db1, chl, idxb,
             outx, outy, outz, sem0, sem1):
    cid = lax.axis_index("c")
    sid = lax.axis_index("s")
    wid = sid * 2 + cid
    b = wid // TPB
    r0 = (wid % TPB) * RPT
    iota16 = lax.iota(jnp.int32, 16)
    z16 = jnp.zeros((16,), jnp.int32)
    inf = jnp.float32(jnp.inf)

    pltpu.sync_copy(xc_hbm.at[b, 0], xl)
    pltpu.sync_copy(yc_hbm.at[b, 0], yl)
    pltpu.sync_copy(zc_hbm.at[b, 0], zl)
    pltpu.sync_copy(thr_hbm.at[b, 0, pl.ds(r0, RPT)], thrl)
    pltpu.sync_copy(cm_hbm.at[b, pl.ds(r0, RPT)], cml)

    def process(i, dbuf):
        rq = r0 + i
        ivecs = jnp.full((16,), i, jnp.int32)
        tv = plsc.load_gather(thrl, [ivecs])
        ncl = jnp.int32(0)
        for cc in range(NCH // 16):
            cmv = plsc.load_gather(cml, [ivecs, cc * 16 + iota16])
            msk = cmv <= tv
            mi = jnp.where(msk, jnp.int32(1), jnp.int32(0))
            cs = plsc.cumsum(mi)
            plsc.store_scatter(chl, [ncl + cs - 1], cc * 16 + iota16, mask=msk)
            ncl = ncl + jnp.sum(mi)
        def chunk_body(jj, cur):
            ckv = plsc.load_gather(chl, [jnp.full((16,), jj, jnp.int32)])
            basev = ckv * CH
            for s in range(CH // 16):
                ivec = basev + (s * 16) + iota16
                v = plsc.load_gather(dbuf, [z16, ivec])
                msk = (v <= tv) & (ivec != rq)
                mi = jnp.where(msk, jnp.int32(1), jnp.int32(0))
                cs = plsc.cumsum(mi)
                plsc.store_scatter(idxb, [cur + cs - 1], ivec, mask=msk)
                cur = cur + plsc.all_reduce_population_count(msk)
            return cur
        curv = lax.fori_loop(0, ncl, chunk_body, jnp.zeros((16,), jnp.int32))
        nsv = jnp.max(curv)
        def merge_body(jj, carry):
            bk, bv = carry
            lane = jj * 16 + iota16
            lm = lane < nsv
            sidx = plsc.load_gather(idxb, [jnp.where(lm, lane, 0)])
            sidx = jnp.where(lm, sidx, jnp.int32(0))
            keys = plsc.load_gather(dbuf, [z16, sidx])
            keys = jnp.where(lm, keys, inf)
            sk, sv = plsc.sort_key_val(keys, sidx)
            rk = lax.rev(sk, (0,))
            rv = lax.rev(sv, (0,))
            cnd = bk <= rk
            mk = jnp.where(cnd, bk, rk)
            mv = jnp.where(cnd, bv, rv)
            nk, nv = plsc.sort_key_val(mk, mv)
            return nk, nv
        bk0 = jnp.full((16,), inf, jnp.float32)
        bv0 = jnp.zeros((16,), jnp.int32)
        _, bv = lax.fori_loop(0, (nsv + 15) // 16, merge_body, (bk0, bv0))
        rqv = r0 + ivecs
        gx = plsc.load_gather(xl, [bv]) - plsc.load_gather(xl, [rqv])
        gy = plsc.load_gather(yl, [bv]) - plsc.load_gather(yl, [rqv])
        gz = plsc.load_gather(zl, [bv]) - plsc.load_gather(zl, [rqv])
        opos = iota16 * RPT + i
        plsc.store_scatter(outx, [opos], gx)
        plsc.store_scatter(outy, [opos], gy)
        plsc.store_scatter(outz, [opos], gz)

    pltpu.make_async_copy(dist_hbm.at[b, pl.ds(r0, 1)], db0, sem0).start()

    def row_pair(k, carry):
        r = r0 + 2 * k
        pltpu.make_async_copy(dist_hbm.at[b, pl.ds(r + 1, 1)], db1, sem1).start()
        pltpu.make_async_copy(dist_hbm.at[b, pl.ds(r, 1)], db0, sem0).wait()
        process(2 * k, db0)

        @pl.when(2 * k + 2 < RPT)
        def _():
            pltpu.make_async_copy(dist_hbm.at[b, pl.ds(r + 2, 1)], db0, sem0).start()
        pltpu.make_async_copy(dist_hbm.at[b, pl.ds(r + 1, 1)], db1, sem1).wait()
        process(2 * k + 1, db1)
        return carry

    lax.fori_loop(0, RPT // 2, row_pair, jnp.int32(0))

    col0 = b * N + r0
    for j in range(KNN):
        pltpu.sync_copy(outx.at[pl.ds(j * RPT, RPT)], ox_hbm.at[j, 0, pl.ds(col0, RPT)])
        pltpu.sync_copy(outy.at[pl.ds(j * RPT, RPT)], oy_hbm.at[j, 0, pl.ds(col0, RPT)])
        pltpu.sync_copy(outz.at[pl.ds(j * RPT, RPT)], oz_hbm.at[j, 0, pl.ds(col0, RPT)])


@functools.cache
def _get_sc_call():
  return pl.kernel(
    _sc_body,
    out_type=[jax.ShapeDtypeStruct((KNN, 1, B * N), jnp.float32)] * 3,
    mesh=plsc.VectorSubcoreMesh(core_axis_name="c", subcore_axis_name="s",
                                num_cores=2, num_subcores=16),
    compiler_params=pltpu.CompilerParams(needs_layout_passes=False),
    scratch_types=[
        pltpu.VMEM((N,), jnp.float32),
        pltpu.VMEM((N,), jnp.float32),
        pltpu.VMEM((N,), jnp.float32),
        pltpu.VMEM((RPT,), jnp.float32),
        pltpu.VMEM((RPT, NCH), jnp.float32),
        pltpu.VMEM((1, N), jnp.float32),
        pltpu.VMEM((1, N), jnp.float32),
        pltpu.VMEM((NCH,), jnp.int32),
        pltpu.VMEM((SVMAX,), jnp.int32),
        pltpu.VMEM((KNN * RPT,), jnp.float32),
        pltpu.VMEM((KNN * RPT,), jnp.float32),
        pltpu.VMEM((KNN * RPT,), jnp.float32),
        pltpu.SemaphoreType.DMA,
        pltpu.SemaphoreType.DMA,
    ],
  )


def _conv_body(x_ref, y_ref, z_ref, w1_ref, w2_ref, w3_ref, o_ref):
    j = pl.program_id(2)
    g = jnp.concatenate([x_ref[0], y_ref[0], z_ref[0]], axis=0)
    h = jnp.dot(w1_ref[...], g, preferred_element_type=jnp.float32)
    h = jnp.maximum(h, 0.0)
    h = jnp.dot(w2_ref[...], h, preferred_element_type=jnp.float32)
    h = jnp.maximum(h, 0.0)
    h = jnp.dot(w3_ref[...], h, preferred_element_type=jnp.float32)

    @pl.when(j == 0)
    def _():
        o_ref[0] = h

    @pl.when(j > 0)
    def _():
        o_ref[0] = jnp.maximum(o_ref[0], h)


_NT = N // NB

_conv_call = pl.pallas_call(
    _conv_body,
    grid=(B, _NT, KNN),
    in_specs=[
        pl.BlockSpec((1, 1, NB), lambda b, t, j: (j, 0, b * _NT + t)),
        pl.BlockSpec((1, 1, NB), lambda b, t, j: (j, 0, b * _NT + t)),
        pl.BlockSpec((1, 1, NB), lambda b, t, j: (j, 0, b * _NT + t)),
        pl.BlockSpec((COUT, 3), lambda b, t, j: (0, 0)),
        pl.BlockSpec((COUT, COUT), lambda b, t, j: (0, 0)),
        pl.BlockSpec((COUT, COUT), lambda b, t, j: (0, 0)),
    ],
    out_specs=pl.BlockSpec((1, COUT, NB), lambda b, t, j: (b, 0, t)),
    out_shape=jax.ShapeDtypeStruct((B, COUT, N), jnp.float32),
)


@jax.jit
def kernel(xyz, W1, W2, W3):
    xyzt = jnp.transpose(xyz, (0, 2, 1))
    dist, cm, thr = _dist_call(xyz, xyzt)
    thr3 = jnp.reshape(thr, (B, 1, N))
    xc = xyzt[:, 0:1, :]
    yc = xyzt[:, 1:2, :]
    zc = xyzt[:, 2:3, :]
    ox, oy, oz = _get_sc_call()(dist, thr3, cm, xc, yc, zc)
    return _conv_call(ox, oy, oz, W1, W2, W3)


# speedup vs baseline: 19.9244x; 1.1612x over previous
import functools

import jax
import jax.numpy as jnp
from jax import lax
from jax.experimental import pallas as pl
from jax.experimental.pallas import tpu as pltpu
from jax.experimental.pallas import tpu_sc as plsc

B = 4
N = 4096
COUT = 128
KSEL = 17
KNN = 16
CH = 64
NCH = N // CH
QT = 256
NB = 2048
NTEC = 32
RPT = (B * N) // NTEC
TPB = N // RPT
SVMAX = N + 64


def _dist_body(q_ref, c_ref, dist_ref, cm_ref, thr_ref):
    q = q_ref[0]
    c = c_ref[0]
    d0 = c[0:1, :] - q[:, 0:1]
    d1 = c[1:2, :] - q[:, 1:2]
    d2 = c[2:3, :] - q[:, 2:3]
    dist = (d0 * d0 + d1 * d1) + d2 * d2
    dist_ref[0] = dist
    cms = [jnp.min(dist[:, i * CH:(i + 1) * CH], axis=1, keepdims=True)
           for i in range(NCH)]
    cm = jnp.concatenate(cms, axis=1)
    cm_ref[0] = cm
    work = cm
    m = None
    for _ in range(KSEL):
        m = jnp.min(work, axis=1, keepdims=True)
        work = jnp.where(work == m, jnp.float32(jnp.inf), work)
    thr_ref[0] = m


_dist_call = pl.pallas_call(
    _dist_body,
    grid=(B, N // QT),
    in_specs=[
        pl.BlockSpec((1, QT, 3), lambda b, t: (b, t, 0)),
        pl.BlockSpec((1, 3, N), lambda b, t: (b, 0, 0)),
    ],
    out_specs=[
        pl.BlockSpec((1, QT, N), lambda b, t: (b, t, 0)),
        pl.BlockSpec((1, QT, NCH), lambda b, t: (b, t, 0)),
        pl.BlockSpec((1, QT, 1), lambda b, t: (b, t, 0)),
    ],
    out_shape=[
        jax.ShapeDtypeStruct((B, N, N), jnp.float32),
        jax.ShapeDtypeStruct((B, N, NCH), jnp.float32),
        jax.ShapeDtypeStruct((B, N, 1), jnp.float32),
    ],
)


def _sc_body(dist_hbm, thr_hbm, cm_hbm, xc_hbm, yc_hbm, zc_hbm,
             ox_hbm, oy_hbm, oz_hbm,
             xl, yl, zl, thrl, cml, db0, db1, chl, idxb,
             outx, outy, outz, sem0, sem1):
    cid = lax.axis_index("c")
    sid = lax.axis_index("s")
    wid = sid * 2 + cid
    b = wid // TPB
    r0 = (wid % TPB) * RPT
    iota16 = lax.iota(jnp.int32, 16)
    z16 = jnp.zeros((16,), jnp.int32)
    inf = jnp.float32(jnp.inf)

    pltpu.sync_copy(xc_hbm.at[b, 0], xl)
    pltpu.sync_copy(yc_hbm.at[b, 0], yl)
    pltpu.sync_copy(zc_hbm.at[b, 0], zl)
    pltpu.sync_copy(thr_hbm.at[b, 0, pl.ds(r0, RPT)], thrl)
    pltpu.sync_copy(cm_hbm.at[b, pl.ds(r0, RPT)], cml)

    def process(i, dbuf):
        rq = r0 + i
        ivecs = jnp.full((16,), i, jnp.int32)
        tv = plsc.load_gather(thrl, [ivecs])
        ncl = jnp.int32(0)
        for cc in range(NCH // 16):
            cmv = plsc.load_gather(cml, [ivecs, cc * 16 + iota16])
            msk = cmv <= tv
            mi = jnp.where(msk, jnp.int32(1), jnp.int32(0))
            cs = plsc.cumsum(mi)
            plsc.store_scatter(chl, [ncl + cs - 1], cc * 16 + iota16, mask=msk)
            ncl = ncl + jnp.sum(mi)
        def chunk_body(jj, cur):
            ckv = plsc.load_gather(chl, [jnp.full((16,), jj, jnp.int32)])
            basev = ckv * CH
            for s in range(CH // 16):
                ivec = basev + (s * 16) + iota16
                v = plsc.load_gather(dbuf, [z16, ivec])
                msk = (v <= tv) & (ivec != rq)
                mi = jnp.where(msk, jnp.int32(1), jnp.int32(0))
                cs = plsc.cumsum(mi)
                plsc.store_scatter(idxb, [cur + cs - 1], ivec, mask=msk)
                cur = cur + plsc.all_reduce_population_count(msk)
            return cur
        curv = lax.fori_loop(0, ncl, chunk_body, jnp.zeros((16,), jnp.int32))
        nsv = jnp.max(curv)
        def merge_body(jj, carry):
            bk, bv = carry
            lane = jj * 16 + iota16
            lm = lane < nsv
            sidx = plsc.load_gather(idxb, [jnp.where(lm, lane, 0)])
            sidx = jnp.where(lm, sidx, jnp.int32(0))
            keys = plsc.load_gather(dbuf, [z16, sidx])
            keys = jnp.where(lm, keys, inf)
            sk, sv = plsc.sort_key_val(keys, sidx)
            rk = lax.rev(sk, (0,))
            rv = lax.rev(sv, (0,))
            cnd = bk <= rk
            mk = jnp.where(cnd, bk, rk)
            mv = jnp.where(cnd, bv, rv)
            nk, nv = plsc.sort_key_val(mk, mv)
            return nk, nv
        bk0 = jnp.full((16,), inf, jnp.float32)
        bv0 = jnp.zeros((16,), jnp.int32)
        _, bv = lax.fori_loop(0, (nsv + 15) // 16, merge_body, (bk0, bv0))
        rqv = r0 + ivecs
        gx = plsc.load_gather(xl, [bv]) - plsc.load_gather(xl, [rqv])
        gy = plsc.load_gather(yl, [bv]) - plsc.load_gather(yl, [rqv])
        gz = plsc.load_gather(zl, [bv]) - plsc.load_gather(zl, [rqv])
        opos = iota16 * RPT + i
        plsc.store_scatter(outx, [opos], gx)
        plsc.store_scatter(outy, [opos], gy)
        plsc.store_scatter(outz, [opos], gz)

    pltpu.make_async_copy(dist_hbm.at[b, pl.ds(r0, 1)], db0, sem0).start()

    def row_pair(k, carry):
        r = r0 + 2 * k
        pltpu.make_async_copy(dist_hbm.at[b, pl.ds(r + 1, 1)], db1, sem1).start()
        pltpu.make_async_copy(dist_hbm.at[b, pl.ds(r, 1)], db0, sem0).wait()
        process(2 * k, db0)

        @pl.when(2 * k + 2 < RPT)
        def _():
            pltpu.make_async_copy(dist_hbm.at[b, pl.ds(r + 2, 1)], db0, sem0).start()
        pltpu.make_async_copy(dist_hbm.at[b, pl.ds(r + 1, 1)], db1, sem1).wait()
        process(2 * k + 1, db1)
        return carry

    lax.fori_loop(0, RPT // 2, row_pair, jnp.int32(0))

    col0 = b * N + r0
    for j in range(KNN):
        pltpu.sync_copy(outx.at[pl.ds(j * RPT, RPT)], ox_hbm.at[j, 0, pl.ds(col0, RPT)])
        pltpu.sync_copy(outy.at[pl.ds(j * RPT, RPT)], oy_hbm.at[j, 0, pl.ds(col0, RPT)])
        pltpu.sync_copy(outz.at[pl.ds(j * RPT, RPT)], oz_hbm.at[j, 0, pl.ds(col0, RPT)])


@functools.cache
def _get_sc_call():
  return pl.kernel(
    _sc_body,
    out_type=[jax.ShapeDtypeStruct((KNN, 1, B * N), jnp.float32)] * 3,
    mesh=plsc.VectorSubcoreMesh(core_axis_name="c", subcore_axis_name="s",
                                num_cores=2, num_subcores=16),
    compiler_params=pltpu.CompilerParams(needs_layout_passes=False),
    scratch_types=[
        pltpu.VMEM((N,), jnp.float32),
        pltpu.VMEM((N,), jnp.float32),
        pltpu.VMEM((N,), jnp.float32),
        pltpu.VMEM((RPT,), jnp.float32),
        pltpu.VMEM((RPT, NCH), jnp.float32),
        pltpu.VMEM((1, N), jnp.float32),
        pltpu.VMEM((1, N), jnp.float32),
        pltpu.VMEM((NCH,), jnp.int32),
        pltpu.VMEM((SVMAX,), jnp.int32),
        pltpu.VMEM((KNN * RPT,), jnp.float32),
        pltpu.VMEM((KNN * RPT,), jnp.float32),
        pltpu.VMEM((KNN * RPT,), jnp.float32),
        pltpu.SemaphoreType.DMA,
        pltpu.SemaphoreType.DMA,
    ],
  )


def _conv_body(x_ref, y_ref, z_ref, w1_ref, w2_ref, w3_ref, o_ref):
    j = pl.program_id(2)
    g = jnp.concatenate([x_ref[0], y_ref[0], z_ref[0]], axis=0)
    h = jnp.dot(w1_ref[...], g, preferred_element_type=jnp.float32)
    h = jnp.maximum(h, 0.0)
    h = jnp.dot(w2_ref[...], h, preferred_element_type=jnp.float32)
    h = jnp.maximum(h, 0.0)
    h = jnp.dot(w3_ref[...], h, preferred_element_type=jnp.float32)

    @pl.when(j == 0)
    def _():
        o_ref[0] = h

    @pl.when(j > 0)
    def _():
        o_ref[0] = jnp.maximum(o_ref[0], h)


_NT = N // NB

_conv_call = pl.pallas_call(
    _conv_body,
    grid=(B, _NT, KNN),
    in_specs=[
        pl.BlockSpec((1, 1, NB), lambda b, t, j: (j, 0, b * _NT + t)),
        pl.BlockSpec((1, 1, NB), lambda b, t, j: (j, 0, b * _NT + t)),
        pl.BlockSpec((1, 1, NB), lambda b, t, j: (j, 0, b * _NT + t)),
        pl.BlockSpec((COUT, 3), lambda b, t, j: (0, 0)),
        pl.BlockSpec((COUT, COUT), lambda b, t, j: (0, 0)),
        pl.BlockSpec((COUT, COUT), lambda b, t, j: (0, 0)),
    ],
    out_specs=pl.BlockSpec((1, COUT, NB), lambda b, t, j: (b, 0, t)),
    out_shape=jax.ShapeDtypeStruct((B, COUT, N), jnp.float32),
)


@jax.jit
def kernel(xyz, W1, W2, W3):
    xyzt = jnp.transpose(xyz, (0, 2, 1))
    dist, cm, thr = _dist_call(xyz, xyzt)
    thr3 = jnp.reshape(thr, (B, 1, N))
    xc = xyzt[:, 0:1, :]
    yc = xyzt[:, 1:2, :]
    zc = xyzt[:, 2:3, :]
    ox, oy, oz = _get_sc_call()(dist, thr3, cm, xc, yc, zc)
    return _conv_call(ox, oy, oz, W1, W2, W3)


# speedup vs baseline: 30.1250x; 1.5120x over previous
import functools

import jax
import jax.numpy as jnp
from jax import lax
from jax.experimental import pallas as pl
from jax.experimental.pallas import tpu as pltpu
from jax.experimental.pallas import tpu_sc as plsc

B = 4
N = 4096
COUT = 128
KSEL = 17
KNN = 16
CH = 32
NCH = N // CH
QT = 256
NB = 2048
NTEC = 32
SVMAX = N + 64


def _dist_body(q_ref, c_ref, dist_ref, cm_ref, thr_ref):
    q = q_ref[0]
    c = c_ref[0]
    d0 = c[0:1, :] - q[:, 0:1]
    d1 = c[1:2, :] - q[:, 1:2]
    d2 = c[2:3, :] - q[:, 2:3]
    dist = (d0 * d0 + d1 * d1) + d2 * d2
    dist_ref[0] = dist
    cms = [jnp.min(dist[:, i * CH:(i + 1) * CH], axis=1, keepdims=True)
           for i in range(NCH)]
    cm = jnp.concatenate(cms, axis=1)
    cm_ref[0] = cm
    work = cm
    m = None
    for _ in range(KSEL):
        m = jnp.min(work, axis=1, keepdims=True)
        work = jnp.where(work == m, jnp.float32(jnp.inf), work)
    thr_ref[0] = m


def _sc_rows(rpt, dist_hbm, thr_hbm, cm_hbm, xc_hbm, yc_hbm, zc_hbm,
             ox_hbm, oy_hbm, oz_hbm,
             xl, yl, zl, thrl, cml, db0, db1, chl, idxb,
             outx, outy, outz, sem0, sem1):
    cid = lax.axis_index("c")
    sid = lax.axis_index("s")
    wid = sid * 2 + cid
    g0 = wid * rpt
    b = g0 // N
    r0 = g0 % N
    iota16 = lax.iota(jnp.int32, 16)
    z16 = jnp.zeros((16,), jnp.int32)
    inf = jnp.float32(jnp.inf)

    pltpu.sync_copy(xc_hbm.at[b, 0], xl)
    pltpu.sync_copy(yc_hbm.at[b, 0], yl)
    pltpu.sync_copy(zc_hbm.at[b, 0], zl)
    pltpu.sync_copy(thr_hbm.at[b, 0, pl.ds(r0, rpt)], thrl)
    pltpu.sync_copy(cm_hbm.at[b, pl.ds(r0, rpt)], cml)

    def process(i, dbuf, sr):
        srv = jnp.full((16,), sr, jnp.int32)
        rq = r0 + i
        ivecs = jnp.full((16,), i, jnp.int32)
        tv = plsc.load_gather(thrl, [ivecs])
        nclv = jnp.zeros((16,), jnp.int32)
        for cc in range(NCH // 16):
            cmv = plsc.load_gather(cml, [ivecs, cc * 16 + iota16])
            msk = cmv <= tv
            mi = jnp.where(msk, jnp.int32(1), jnp.int32(0))
            cs = plsc.cumsum(mi)
            plsc.store_scatter(chl, [nclv + cs - 1], cc * 16 + iota16, mask=msk)
            nclv = nclv + plsc.all_reduce_population_count(msk)
        ncl = jnp.max(nclv)
        def chunk_body(jj, cur):
            ckv = plsc.load_gather(chl, [jnp.full((16,), jj, jnp.int32)])
            basev = ckv * CH
            for s in range(CH // 16):
                ivec = basev + (s * 16) + iota16
                v = plsc.load_gather(dbuf, [srv, ivec])
                msk = (v <= tv) & (ivec != rq)
                mi = jnp.where(msk, jnp.int32(1), jnp.int32(0))
                cs = plsc.cumsum(mi)
                plsc.store_scatter(idxb, [cur + cs - 1], ivec, mask=msk)
                cur = cur + plsc.all_reduce_population_count(msk)
            return cur
        curv = lax.fori_loop(0, ncl, chunk_body, jnp.zeros((16,), jnp.int32))
        nsv = jnp.max(curv)
        def merge_body(jj, carry):
            bk, bv = carry
            lane = jj * 16 + iota16
            lm = lane < nsv
            sidx = plsc.load_gather(idxb, [jnp.where(lm, lane, 0)])
            sidx = jnp.where(lm, sidx, jnp.int32(0))
            keys = plsc.load_gather(dbuf, [srv, sidx])
            keys = jnp.where(lm, keys, inf)
            sk, sv = plsc.sort_key_val(keys, sidx)
            rk = lax.rev(sk, (0,))
            rv = lax.rev(sv, (0,))
            cnd = bk <= rk
            mk = jnp.where(cnd, bk, rk)
            mv = jnp.where(cnd, bv, rv)
            nk, nv = plsc.sort_key_val(mk, mv)
            return nk, nv
        bk0 = jnp.full((16,), inf, jnp.float32)
        bv0 = jnp.zeros((16,), jnp.int32)
        _, bv = lax.fori_loop(0, (nsv + 15) // 16, merge_body, (bk0, bv0))
        rqv = r0 + ivecs
        gx = plsc.load_gather(xl, [bv]) - plsc.load_gather(xl, [rqv])
        gy = plsc.load_gather(yl, [bv]) - plsc.load_gather(yl, [rqv])
        gz = plsc.load_gather(zl, [bv]) - plsc.load_gather(zl, [rqv])
        opos = iota16 * rpt + i
        plsc.store_scatter(outx, [opos], gx)
        plsc.store_scatter(outy, [opos], gy)
        plsc.store_scatter(outz, [opos], gz)

    pltpu.make_async_copy(dist_hbm.at[b, pl.ds(r0, 1)], db0, sem0).start()

    def row_pair(k, carry):
        r = r0 + 2 * k
        pltpu.make_async_copy(dist_hbm.at[b, pl.ds(r + 1, 1)], db1, sem1).start()
        pltpu.make_async_copy(dist_hbm.at[b, pl.ds(r, 1)], db0, sem0).wait()
        process(2 * k, db0, 0)

        @pl.when(2 * k + 2 < rpt)
        def _():
            pltpu.make_async_copy(dist_hbm.at[b, pl.ds(r + 2, 1)], db0, sem0).start()
        pltpu.make_async_copy(dist_hbm.at[b, pl.ds(r + 1, 1)], db1, sem1).wait()
        process(2 * k + 1, db1, 0)
        return carry

    lax.fori_loop(0, rpt // 2, row_pair, jnp.int32(0))

    col0 = b * N + r0
    for j in range(KNN):
        pltpu.sync_copy(outx.at[pl.ds(j * rpt, rpt)], ox_hbm.at[j, 0, pl.ds(col0, rpt)])
        pltpu.sync_copy(outy.at[pl.ds(j * rpt, rpt)], oy_hbm.at[j, 0, pl.ds(col0, rpt)])
        pltpu.sync_copy(outz.at[pl.ds(j * rpt, rpt)], oz_hbm.at[j, 0, pl.ds(col0, rpt)])


RPT1 = N // NTEC


def _sc_body_1(*args):
    _sc_rows(RPT1, *args)


def _conv_body(x_ref, y_ref, z_ref, w1_ref, w2_ref, w3_ref, o_ref):
    j = pl.program_id(2)
    g = jnp.concatenate([x_ref[0], y_ref[0], z_ref[0]], axis=0)
    h = jnp.dot(w1_ref[...], g, preferred_element_type=jnp.float32)
    h = jnp.maximum(h, 0.0)
    h = jnp.dot(w2_ref[...], h, preferred_element_type=jnp.float32)
    h = jnp.maximum(h, 0.0)
    h = jnp.dot(w3_ref[...], h, preferred_element_type=jnp.float32)

    @pl.when(j == 0)
    def _():
        o_ref[0] = h

    @pl.when(j > 0)
    def _():
        o_ref[0] = jnp.maximum(o_ref[0], h)


_NT = N // NB

_dist_call_1 = pl.pallas_call(
    _dist_body,
    grid=(1, N // QT),
    in_specs=[
        pl.BlockSpec((1, QT, 3), lambda b, t: (b, t, 0)),
        pl.BlockSpec((1, 3, N), lambda b, t: (b, 0, 0)),
    ],
    out_specs=[
        pl.BlockSpec((1, QT, N), lambda b, t: (b, t, 0)),
        pl.BlockSpec((1, QT, NCH), lambda b, t: (b, t, 0)),
        pl.BlockSpec((1, QT, 1), lambda b, t: (b, t, 0)),
    ],
    out_shape=[
        jax.ShapeDtypeStruct((1, N, N), jnp.float32),
        jax.ShapeDtypeStruct((1, N, NCH), jnp.float32),
        jax.ShapeDtypeStruct((1, N, 1), jnp.float32),
    ],
)


@functools.cache
def _get_sc_call_1():
  return pl.kernel(
    _sc_body_1,
    out_type=[jax.ShapeDtypeStruct((KNN, 1, N), jnp.float32)] * 3,
    mesh=plsc.VectorSubcoreMesh(core_axis_name="c", subcore_axis_name="s",
                                num_cores=2, num_subcores=16),
    compiler_params=pltpu.CompilerParams(needs_layout_passes=False),
    scratch_types=[
        pltpu.VMEM((N,), jnp.float32),
        pltpu.VMEM((N,), jnp.float32),
        pltpu.VMEM((N,), jnp.float32),
        pltpu.VMEM((RPT1,), jnp.float32),
        pltpu.VMEM((RPT1, NCH), jnp.float32),
        pltpu.VMEM((1, N), jnp.float32),
        pltpu.VMEM((1, N), jnp.float32),
        pltpu.VMEM((NCH,), jnp.int32),
        pltpu.VMEM((SVMAX,), jnp.int32),
        pltpu.VMEM((KNN * RPT1,), jnp.float32),
        pltpu.VMEM((KNN * RPT1,), jnp.float32),
        pltpu.VMEM((KNN * RPT1,), jnp.float32),
        pltpu.SemaphoreType.DMA,
        pltpu.SemaphoreType.DMA,
    ],
  )


_conv_call_1 = pl.pallas_call(
    _conv_body,
    grid=(1, _NT, KNN),
    in_specs=[
        pl.BlockSpec((1, 1, NB), lambda b, t, j: (j, 0, b * _NT + t)),
        pl.BlockSpec((1, 1, NB), lambda b, t, j: (j, 0, b * _NT + t)),
        pl.BlockSpec((1, 1, NB), lambda b, t, j: (j, 0, b * _NT + t)),
        pl.BlockSpec((COUT, 3), lambda b, t, j: (0, 0)),
        pl.BlockSpec((COUT, COUT), lambda b, t, j: (0, 0)),
        pl.BlockSpec((COUT, COUT), lambda b, t, j: (0, 0)),
    ],
    out_specs=pl.BlockSpec((1, COUT, NB), lambda b, t, j: (b, 0, t)),
    out_shape=jax.ShapeDtypeStruct((1, COUT, N), jnp.float32),
)


@jax.jit
def kernel(xyz, W1, W2, W3):
    xyzt = jnp.transpose(xyz, (0, 2, 1))
    outs = []
    for b in range(B):
        xb = lax.slice_in_dim(xyz, b, b + 1, axis=0)
        xtb = lax.slice_in_dim(xyzt, b, b + 1, axis=0)
        dist, cm, thr = _dist_call_1(xb, xtb)
        thr3 = jnp.reshape(thr, (1, 1, N))
        ox, oy, oz = _get_sc_call_1()(dist, thr3, cm,
                                      xtb[:, 0:1, :], xtb[:, 1:2, :],
                                      xtb[:, 2:3, :])
        outs.append(_conv_call_1(ox, oy, oz, W1, W2, W3))
    return jnp.concatenate(outs, axis=0)
